# baseline (speedup 1.0000x reference)
import functools

import jax
import jax.numpy as jnp
from jax import lax
from jax.experimental import pallas as pl
from jax.experimental.pallas import tpu as pltpu
from jax.experimental.pallas import tpu_sc as plsc

_NC, _NS, _L = 2, 16, 16
_NW = _NC * _NS
_CHUNK = 12800


def kernel(x):
    n = x.size
    per_w = n // _NW
    nchunks = per_w // _CHUNK
    assert per_w % _CHUNK == 0

    mesh = plsc.VectorSubcoreMesh(core_axis_name="c", subcore_axis_name="s")

    @functools.partial(
        pl.kernel, mesh=mesh,
        out_type=jax.ShapeDtypeStruct((n,), jnp.float32),
        scratch_types=[
            pltpu.VMEM((_CHUNK,), jnp.float32),
            pltpu.VMEM((_CHUNK,), jnp.float32),
        ],
    )
    def sc_sigmoid(x_hbm, o_hbm, buf_in, buf_out):
        wid = lax.axis_index("s") * _NC + lax.axis_index("c")
        base = wid * per_w
        for c in range(nchunks):
            off = base + c * _CHUNK
            pltpu.sync_copy(x_hbm.at[pl.ds(off, _CHUNK)], buf_in)

            def body(j, carry):
                v = buf_in[pl.ds(j * _L, _L)]
                buf_out[pl.ds(j * _L, _L)] = 1.0 / (1.0 + jnp.exp(-v))
                return carry

            lax.fori_loop(0, _CHUNK // _L, body, 0)
            pltpu.sync_copy(buf_out, o_hbm.at[pl.ds(off, _CHUNK)])

    xt_flat = x.T.reshape(-1)
    out_flat = sc_sigmoid(xt_flat)
    return out_flat.reshape(x.shape[1], x.shape[0]).T


# speedup vs baseline: 8.0597x; 8.0597x over previous
import jax
import jax.numpy as jnp
from jax.experimental import pallas as pl
from jax.experimental.pallas import tpu as pltpu

_DEPTH = 8
_CHUNK_ROWS = [8] * 5 + [40] * 23 + [8] * 5
_MAX_ROWS = max(_CHUNK_ROWS)
_OFFSETS = [sum(_CHUNK_ROWS[:i]) for i in range(len(_CHUNK_ROWS))]
assert sum(_CHUNK_ROWS) == 1000


def _sigmoid_stream(x_hbm, o_hbm, in_buf, out_buf, load_sems, store_sems):
    nchunks = len(_CHUNK_ROWS)

    def load(i, slot):
        r = _CHUNK_ROWS[i]
        return pltpu.make_async_copy(
            x_hbm.at[pl.ds(_OFFSETS[i], r), :],
            in_buf.at[slot, pl.ds(0, r)], load_sems.at[slot])

    def store(i, slot):
        r = _CHUNK_ROWS[i]
        return pltpu.make_async_copy(
            out_buf.at[slot, pl.ds(0, r)],
            o_hbm.at[pl.ds(_OFFSETS[i], r), :], store_sems.at[slot])

    for k in range(min(_DEPTH, nchunks)):
        load(k, k).start()

    for i in range(nchunks):
        slot = i % _DEPTH
        r = _CHUNK_ROWS[i]
        load(i, slot).wait()
        if i >= _DEPTH:
            store(i - _DEPTH, slot).wait()
        out_buf[slot, :r] = 0.5 * jnp.tanh(0.5 * in_buf[slot, :r]) + 0.5
        store(i, slot).start()
        if i + _DEPTH < nchunks:
            load(i + _DEPTH, slot).start()

    for i in range(max(nchunks - _DEPTH, 0), nchunks):
        store(i, i % _DEPTH).wait()


def kernel(x):
    xt = x.T
    rows, cols = xt.shape
    out_t = pl.pallas_call(
        _sigmoid_stream,
        out_shape=jax.ShapeDtypeStruct((rows, cols), x.dtype),
        in_specs=[pl.BlockSpec(memory_space=pl.ANY)],
        out_specs=pl.BlockSpec(memory_space=pl.ANY),
        scratch_shapes=[
            pltpu.VMEM((_DEPTH, _MAX_ROWS, cols), x.dtype),
            pltpu.VMEM((_DEPTH, _MAX_ROWS, cols), x.dtype),
            pltpu.SemaphoreType.DMA((_DEPTH,)),
            pltpu.SemaphoreType.DMA((_DEPTH,)),
        ],
    )(xt)
    return out_t.T
